# baseline (speedup 1.0000x reference)
import functools

import jax
import jax.numpy as jnp
from jax import lax
from jax.experimental import pallas as pl
from jax.experimental.pallas import tpu as pltpu
from jax.experimental.pallas import tpu_sc as plsc

NBINS = 512
MAX_VAL = 2.0
MIN_VAL = -2.0
EPS = 1e-06
EMP_THRESH = 0.008
BIG = 99999.0
BW = (MAX_VAL - MIN_VAL) / NBINS
INV_BW = 1.0 / BW
S_TH_B = 0.99609375
SENT = 1.0 + EPS - 0.5 * BIG * BIG

TM = 2048
TN = 2048

N_WORKERS = 16
LANES = 16


def _stage1_kernel(a_ref, p_ref, hv_ref, ma_ref, mp128_ref, pos_ref):
    i = pl.program_id(0)
    j = pl.program_id(1)
    ni = pl.num_programs(0)
    nj = pl.num_programs(1)
    a = a_ref[...].astype(jnp.bfloat16)
    p = p_ref[...].astype(jnp.bfloat16)
    s32 = jax.lax.dot_general(
        a, p, (((1,), (1,)), ((), ())),
        preferred_element_type=jnp.float32,
    )
    s = s32.astype(jnp.bfloat16)
    sent = jnp.bfloat16(SENT)
    thr = jnp.where(s >= jnp.bfloat16(S_TH_B), sent, s)

    def _reduce_and_store(m):
        colmax = jnp.max(m, axis=0, keepdims=True)
        rs = m[:, 0:128]
        for k in range(1, TN // 128):
            rs = jnp.maximum(rs, m[:, k * 128:(k + 1) * 128])

        @pl.when(i == 0)
        def _():
            ma_ref[0:1, pl.ds(j * TN, TN)] = colmax

        @pl.when(i > 0)
        def _():
            ma_ref[0:1, pl.ds(j * TN, TN)] = jnp.maximum(
                ma_ref[0:1, pl.ds(j * TN, TN)], colmax)

        @pl.when(j == 0)
        def _():
            mp128_ref[pl.ds(i * TM, TM), :] = rs

        @pl.when(j > 0)
        def _():
            mp128_ref[pl.ds(i * TM, TM), :] = jnp.maximum(
                mp128_ref[pl.ds(i * TM, TM), :], rs)

    @pl.when(i == j)
    def _():
        eq = (jax.lax.broadcasted_iota(jnp.int32, (TM, TN), 0)
              == jax.lax.broadcasted_iota(jnp.int32, (TM, TN), 1))
        d = jnp.sum(jnp.where(eq, s32, 0.0), axis=0, keepdims=True)
        pos_ref[0:1, pl.ds(j * TN, TN)] = d
        _reduce_and_store(jnp.where(eq, sent, thr))

    @pl.when(i != j)
    def _():
        _reduce_and_store(thr)

    @pl.when((i == ni - 1) & (j == nj - 1))
    def _():
        rowmax = jnp.max(mp128_ref[...], axis=1)
        neg_dot = jnp.maximum(ma_ref[...],
                              rowmax.reshape(1, -1)).astype(jnp.float32)
        neg = jnp.sqrt((1.0 - neg_dot + EPS) * 2.0)
        pos = jnp.sqrt((1.0 - pos_ref[...] + EPS) * 2.0)
        hv_ref[...] = pos - neg


def _stage2_sc_kernel(hv_hbm, out_hbm, hv_v, idx_v, val_v, bidx_v,
                      hist_v, cdf_v, acc_v, part_v, out_v,
                      hist_sh, part_sh):
    cid = lax.axis_index("c")
    wid = lax.axis_index("s")
    vpw = hv_v.shape[0]
    nchunk = vpw // LANES

    @pl.when(cid == 0)
    def _():
        pltpu.sync_copy(hv_hbm.at[pl.ds(wid * vpw, vpw)], hv_v)

        @pl.when(wid == 0)
        def _():
            for c in range(NBINS // LANES):
                hist_v[pl.ds(c * LANES, LANES)] = jnp.zeros(
                    (LANES,), jnp.float32)
            pltpu.sync_copy(hist_v, hist_sh)

        plsc.subcore_barrier()

        for c in range(nchunk):
            v = hv_v[pl.ds(c * LANES, LANES)]
            t = (v - MIN_VAL) * INV_BW
            lo = t.astype(jnp.int32)
            lo = jnp.where(lo.astype(jnp.float32) > t, lo - 1, lo)
            lo_f = lo.astype(jnp.float32)
            alpha = 1.0 - (t - lo_f)
            hi = jnp.minimum(jnp.maximum(lo + 1, 0), NBINS - 1)
            in_lo = (lo >= 0) & (lo <= NBINS - 1)
            idx_v[pl.ds(c * LANES, LANES)] = jnp.where(in_lo, lo, 0)
            val_v[pl.ds(c * LANES, LANES)] = jnp.where(in_lo, alpha, 0.0)
            idx_v[pl.ds(vpw + c * LANES, LANES)] = hi
            val_v[pl.ds(vpw + c * LANES, LANES)] = 1.0 - alpha
            bidx_v[pl.ds(c * LANES, LANES)] = jnp.minimum(
                jnp.maximum(lo, 0), NBINS - 1)

        pltpu.sync_copy(val_v, hist_sh.at[idx_v], add=True)
        plsc.subcore_barrier()

        pltpu.sync_copy(hist_sh, hist_v)
        run_vec = jnp.zeros((LANES,), jnp.float32)
        for c in range(NBINS // LANES):
            h16 = hist_v[pl.ds(c * LANES, LANES)]
            cdf_v[pl.ds(c * LANES, LANES)] = plsc.cumsum(h16) + run_vec
            run_vec = run_vec + jnp.broadcast_to(jnp.sum(h16), (LANES,))

        acc = jnp.zeros((LANES,), jnp.float32)
        for c in range(nchunk):
            v = hv_v[pl.ds(c * LANES, LANES)]
            b16 = bidx_v[pl.ds(c * LANES, LANES)]
            w = plsc.load_gather(cdf_v, [b16])
            acc = acc + v * w
        acc_v[...] = acc
        pltpu.sync_copy(acc_v, part_sh.at[pl.ds(wid * LANES, LANES)])
        plsc.subcore_barrier()

        @pl.when(wid == 0)
        def _():
            pltpu.sync_copy(part_sh, part_v)
            tot = jnp.zeros((LANES,), jnp.float32)
            for r in range(N_WORKERS):
                tot = tot + part_v[pl.ds(r * LANES, LANES)]
            t_vec = jnp.broadcast_to(jnp.sum(tot), (LANES,))
            out_v[...] = t_vec / (run_vec * jnp.float32(N_WORKERS * vpw))
            pltpu.sync_copy(out_v, out_hbm)


def kernel(x, histogram):
    n = x.shape[0] // 2
    a = x[:n]
    p = x[n:]
    grid = (n // TM, n // TN)
    hv = pl.pallas_call(
        _stage1_kernel,
        grid=grid,
        in_specs=[
            pl.BlockSpec((TM, x.shape[1]), lambda i, j: (i, 0)),
            pl.BlockSpec((TN, x.shape[1]), lambda i, j: (j, 0)),
        ],
        out_specs=pl.BlockSpec((1, n), lambda i, j: (0, 0)),
        out_shape=jax.ShapeDtypeStruct((1, n), jnp.float32),
        scratch_shapes=[
            pltpu.VMEM((1, n), jnp.bfloat16),
            pltpu.VMEM((n, 128), jnp.bfloat16),
            pltpu.VMEM((1, n), jnp.float32),
        ],
    )(a, p)

    vpw = n // N_WORKERS
    sc_fn = pl.kernel(
        _stage2_sc_kernel,
        out_type=jax.ShapeDtypeStruct((LANES,), jnp.float32),
        mesh=plsc.VectorSubcoreMesh(
            core_axis_name="c", subcore_axis_name="s",
            num_cores=2, num_subcores=N_WORKERS),
        compiler_params=pltpu.CompilerParams(needs_layout_passes=False),
        scratch_types=[
            pltpu.VMEM((vpw,), jnp.float32),
            pltpu.VMEM((2 * vpw,), jnp.int32),
            pltpu.VMEM((2 * vpw,), jnp.float32),
            pltpu.VMEM((vpw,), jnp.int32),
            pltpu.VMEM((NBINS,), jnp.float32),
            pltpu.VMEM((NBINS,), jnp.float32),
            pltpu.VMEM((LANES,), jnp.float32),
            pltpu.VMEM((N_WORKERS * LANES,), jnp.float32),
            pltpu.VMEM((LANES,), jnp.float32),
            pltpu.VMEM_SHARED((NBINS,), jnp.float32),
            pltpu.VMEM_SHARED((N_WORKERS * LANES,), jnp.float32),
        ],
    )
    loss_vec = sc_fn(hv.reshape(n))
    return loss_vec[0]
